# baseline (speedup 1.0000x reference)
import jax
import jax.numpy as jnp
from jax.experimental import pallas as pl
from jax.experimental.pallas import tpu as pltpu
from jax.experimental.pallas import tpu_sc as plsc

_EMB = 32
_SCALE = float(_EMB) ** 0.5
_LANES = 16
_ROWS_PER_STEP = 8


def _gather_scale(x, table):
    n_rows, row_len = x.shape
    win = _ROWS_PER_STEP * row_len
    total = n_rows * row_len
    mesh = plsc.VectorSubcoreMesh(core_axis_name="c", subcore_axis_name="s")

    @pl.kernel(
        out_type=jax.ShapeDtypeStruct((total, _EMB), jnp.float32),
        mesh=mesh,
        compiler_params=pltpu.CompilerParams(use_tc_tiling_on_sc=False),
    )
    def k(x_hbm, table_hbm, out_hbm):
        def body(idx_vmem, out_vmem):
            for r in range(_ROWS_PER_STEP):
                pltpu.sync_copy(
                    table_hbm.at[idx_vmem.at[r]],
                    out_vmem.at[pl.ds(r * row_len, row_len)],
                )

            @pl.loop(0, win)
            def _(j):
                for c in range(_EMB // _LANES):
                    sl = (pl.ds(j, 1), pl.ds(c * _LANES, _LANES))
                    out_vmem.at[sl][...] = out_vmem.at[sl][...] * _SCALE

        pltpu.emit_pipeline(
            body,
            grid=(n_rows // _ROWS_PER_STEP,),
            in_specs=[pl.BlockSpec((_ROWS_PER_STEP, row_len), lambda i: (i, 0))],
            out_specs=[pl.BlockSpec((win, _EMB), lambda i: (i, 0))],
            core_axis_name=("c", "s"),
            dimension_semantics=(pltpu.PARALLEL,),
        )(x_hbm, out_hbm)

    return k(x, table)


def kernel(x, table):
    b0, b1 = x.shape
    out = _gather_scale(x.astype(jnp.int32), table)
    return out.reshape(b0, b1, _EMB)


# speedup vs baseline: 1.2436x; 1.2436x over previous
import jax
import jax.numpy as jnp
from jax.experimental import pallas as pl
from jax.experimental.pallas import tpu as pltpu
from jax.experimental.pallas import tpu_sc as plsc

_EMB = 32
_SCALE = float(_EMB) ** 0.5
_LANES = 16
_ROWS_PER_STEP = 8


def _gather_scale(x, table):
    n_rows, row_len = x.shape
    win = _ROWS_PER_STEP * row_len
    total = n_rows * row_len
    mesh = plsc.VectorSubcoreMesh(core_axis_name="c", subcore_axis_name="s")

    @pl.kernel(
        out_type=jax.ShapeDtypeStruct((total, _EMB), jnp.float32),
        mesh=mesh,
        compiler_params=pltpu.CompilerParams(use_tc_tiling_on_sc=False),
    )
    def k(x_hbm, table_hbm, out_hbm):
        def body(idx_vmem, out_vmem):
            for r in range(_ROWS_PER_STEP):
                pltpu.sync_copy(
                    table_hbm.at[idx_vmem.at[r]],
                    out_vmem.at[pl.ds(r * row_len, row_len)],
                )

            @plsc.parallel_loop(0, win, unroll=8)
            def _(j):
                for c in range(_EMB // _LANES):
                    sl = (pl.ds(j, 1), pl.ds(c * _LANES, _LANES))
                    out_vmem.at[sl][...] = out_vmem.at[sl][...] * _SCALE

        pltpu.emit_pipeline(
            body,
            grid=(n_rows // _ROWS_PER_STEP,),
            in_specs=[pl.BlockSpec((_ROWS_PER_STEP, row_len), lambda i: (i, 0))],
            out_specs=[pl.BlockSpec((win, _EMB), lambda i: (i, 0))],
            core_axis_name=("c", "s"),
            dimension_semantics=(pltpu.PARALLEL,),
        )(x_hbm, out_hbm)

    return k(x, table)


def kernel(x, table):
    b0, b1 = x.shape
    if x.dtype != jnp.int32:
        x = x.astype(jnp.int32)
    out = _gather_scale(x, table)
    return out.reshape(b0, b1, _EMB)


# speedup vs baseline: 1.3582x; 1.0922x over previous
import jax
import jax.numpy as jnp
from jax.experimental import pallas as pl
from jax.experimental.pallas import tpu as pltpu
from jax.experimental.pallas import tpu_sc as plsc

_EMB = 32
_SCALE = float(_EMB) ** 0.5
_LANES = 16
_WINDOW = 1024


def _gather_scale(xt, table):
    n_cols, n_rows = xt.shape
    mesh = plsc.VectorSubcoreMesh(core_axis_name="c", subcore_axis_name="s")

    @pl.kernel(
        out_type=jax.ShapeDtypeStruct((n_cols, n_rows, _EMB), jnp.float32),
        mesh=mesh,
        compiler_params=pltpu.CompilerParams(use_tc_tiling_on_sc=False),
    )
    def k(xt_hbm, table_hbm, out_hbm):
        def body(idx_vmem, out_vmem):
            rows = out_vmem.at[0]
            pltpu.sync_copy(table_hbm.at[idx_vmem.at[0]], rows)

            @plsc.parallel_loop(0, _WINDOW, unroll=8)
            def _(j):
                for c in range(_EMB // _LANES):
                    sl = (pl.ds(j, 1), pl.ds(c * _LANES, _LANES))
                    rows.at[sl][...] = rows.at[sl][...] * _SCALE

        pltpu.emit_pipeline(
            body,
            grid=(n_cols, n_rows // _WINDOW),
            in_specs=[pl.BlockSpec((1, _WINDOW), lambda j, i: (j, i))],
            out_specs=[pl.BlockSpec((1, _WINDOW, _EMB), lambda j, i: (j, i, 0))],
            core_axis_name=("c", "s"),
            dimension_semantics=(pltpu.PARALLEL, pltpu.PARALLEL),
        )(xt_hbm, out_hbm)

    return k(xt, table)


def kernel(x, table):
    b0, b1 = x.shape
    if x.dtype != jnp.int32:
        x = x.astype(jnp.int32)
    out_t = _gather_scale(x.T, table)
    return jnp.transpose(out_t, (1, 0, 2))
